# Initial kernel scaffold; baseline (speedup 1.0000x reference)
import jax
import jax.numpy as jnp
from jax.experimental import pallas as pl


def kernel(x, edge_index, W, b):
    raise NotImplementedError("write your pallas kernel here")


# speedup vs baseline: 16.0177x; 16.0177x over previous
import functools

import jax
import jax.numpy as jnp
from jax import lax
from jax.experimental import pallas as pl
from jax.experimental.pallas import tpu as pltpu
from jax.experimental.pallas import tpu_sc as plsc

N_NODES = 10000
N_EDGES = 320000
D = 128

NC = 2
NS = 16
NW = NC * NS
N_PAD = 10240
E_PAD = 327680
E_PER_W = E_PAD // NW
CHUNK = 128
N_CHUNKS = E_PER_W // CHUNK
HALF = N_CHUNKS // 2
ROWS_PER_TILE = N_PAD // NS


def _sc_mesh():
    return plsc.VectorSubcoreMesh(core_axis_name="c", subcore_axis_name="s")


def _sc_hist(dst3, zeros1):
    @functools.partial(
        pl.kernel,
        out_type=jax.ShapeDtypeStruct((NC, N_PAD), jnp.float32),
        mesh=_sc_mesh(),
        scratch_types=[
            pltpu.VMEM((N_CHUNKS, CHUNK), jnp.int32),
            pltpu.VMEM((CHUNK,), jnp.float32),
            pltpu.VMEM_SHARED((N_PAD,), jnp.float32),
        ],
    )
    def hist_kernel(dst_hbm, zeros_hbm, out_hbm, idx_v, ones_v, deg_sh):
        c = lax.axis_index("c")
        s = lax.axis_index("s")
        wid = c * NS + s
        pltpu.sync_copy(
            zeros_hbm.at[pl.ds(s * ROWS_PER_TILE, ROWS_PER_TILE)],
            deg_sh.at[pl.ds(s * ROWS_PER_TILE, ROWS_PER_TILE)],
        )
        pltpu.sync_copy(dst_hbm.at[wid], idx_v)
        for i in range(CHUNK // 16):
            ones_v[pl.ds(i * 16, 16)] = jnp.ones((16,), jnp.float32)
        plsc.subcore_barrier()

        def body(j, carry):
            pltpu.sync_copy(ones_v, deg_sh.at[idx_v.at[j]], add=True)
            return carry

        lax.fori_loop(0, N_CHUNKS, body, 0)
        plsc.subcore_barrier()
        pltpu.sync_copy(
            deg_sh.at[pl.ds(s * ROWS_PER_TILE, ROWS_PER_TILE)],
            out_hbm.at[c, pl.ds(s * ROWS_PER_TILE, ROWS_PER_TILE)],
        )

    return hist_kernel(dst3, zeros1)


def _tc_scale_matmul(x_pad, W, degT):
    blk = 1024
    grid = N_PAD // blk

    def body(x_ref, w_ref, dp_ref, o_ref):
        deg = jnp.sum(dp_ref[...], axis=1, keepdims=True) + 1.0
        dis = lax.rsqrt(deg)
        h = jnp.dot(x_ref[...], w_ref[...], preferred_element_type=jnp.float32)
        o_ref[...] = dis * h

    return pl.pallas_call(
        body,
        grid=(grid,),
        in_specs=[
            pl.BlockSpec((blk, D), lambda i: (i, 0)),
            pl.BlockSpec((D, D), lambda i: (0, 0)),
            pl.BlockSpec((blk, NC), lambda i: (i, 0)),
        ],
        out_specs=pl.BlockSpec((blk, D), lambda i: (i, 0)),
        out_shape=jax.ShapeDtypeStruct((N_PAD, D), jnp.float32),
    )(x_pad, W, degT)


def _sc_agg(hp, src3, dst3, zeros2):
    @functools.partial(
        pl.kernel,
        out_type=jax.ShapeDtypeStruct((NC, N_PAD, D), jnp.float32),
        mesh=_sc_mesh(),
        scratch_types=[
            pltpu.VMEM((HALF, CHUNK), jnp.int32),
            pltpu.VMEM((HALF, CHUNK), jnp.int32),
            pltpu.VMEM((2, CHUNK, D), jnp.float32),
            pltpu.VMEM_SHARED((N_PAD, D), jnp.float32),
            pltpu.SemaphoreType.DMA,
            pltpu.SemaphoreType.DMA,
        ],
    )
    def agg_kernel(hp_hbm, src_hbm, dst_hbm, zeros_hbm, out_hbm,
                   src_v, dst_v, rows_v, agg_sh, sem0, sem1):
        c = lax.axis_index("c")
        s = lax.axis_index("s")
        wid = c * NS + s
        pltpu.sync_copy(
            zeros_hbm.at[pl.ds(s * ROWS_PER_TILE, ROWS_PER_TILE)],
            agg_sh.at[pl.ds(s * ROWS_PER_TILE, ROWS_PER_TILE)],
        )
        plsc.subcore_barrier()

        sems = (sem0, sem1)
        for half in range(2):
            pltpu.sync_copy(src_hbm.at[wid, pl.ds(half * HALF, HALF)], src_v)
            pltpu.sync_copy(dst_hbm.at[wid, pl.ds(half * HALF, HALF)], dst_v)
            pltpu.async_copy(hp_hbm.at[src_v.at[0]], rows_v.at[0], sem0)
            pltpu.async_copy(hp_hbm.at[src_v.at[1]], rows_v.at[1], sem1)

            def pair(jb, carry):
                for buf in range(2):
                    j = 2 * jb + buf
                    pltpu.make_async_copy(hp_hbm.at[src_v.at[j]],
                                          rows_v.at[buf], sems[buf]).wait()
                    pltpu.sync_copy(rows_v.at[buf], agg_sh.at[dst_v.at[j]],
                                    add=True)
                    pltpu.async_copy(hp_hbm.at[src_v.at[j + 2]],
                                     rows_v.at[buf], sems[buf])
                return carry

            lax.fori_loop(0, HALF // 2 - 1, pair, 0)
            for buf in range(2):
                j = HALF - 2 + buf
                pltpu.make_async_copy(hp_hbm.at[src_v.at[j]],
                                      rows_v.at[buf], sems[buf]).wait()
                pltpu.sync_copy(rows_v.at[buf], agg_sh.at[dst_v.at[j]],
                                add=True)

        plsc.subcore_barrier()
        pltpu.sync_copy(
            agg_sh.at[pl.ds(s * ROWS_PER_TILE, ROWS_PER_TILE)],
            out_hbm.at[c, pl.ds(s * ROWS_PER_TILE, ROWS_PER_TILE)],
        )

    return agg_kernel(hp, src3, dst3, zeros2)


def _tc_finish(agg_parts, hp, degT, b2):
    blk = 1024
    grid = N_PAD // blk

    def body(a_ref, h_ref, dp_ref, b_ref, o_ref):
        deg = jnp.sum(dp_ref[...], axis=1, keepdims=True) + 1.0
        dis = lax.rsqrt(deg)
        total = a_ref[0] + a_ref[1] + h_ref[...]
        o_ref[...] = jnp.maximum(dis * total + b_ref[...], 0.0)

    return pl.pallas_call(
        body,
        grid=(grid,),
        in_specs=[
            pl.BlockSpec((NC, blk, D), lambda i: (0, i, 0)),
            pl.BlockSpec((blk, D), lambda i: (i, 0)),
            pl.BlockSpec((blk, NC), lambda i: (i, 0)),
            pl.BlockSpec((1, D), lambda i: (0, 0)),
        ],
        out_specs=pl.BlockSpec((blk, D), lambda i: (i, 0)),
        out_shape=jax.ShapeDtypeStruct((N_PAD, D), jnp.float32),
    )(agg_parts, hp, degT, b2)


def kernel(x, edge_index, W, b):
    src = edge_index[0].astype(jnp.int32)
    dst = edge_index[1].astype(jnp.int32)
    pad = jnp.full((E_PAD - N_EDGES,), N_NODES, jnp.int32)
    src3 = jnp.concatenate([src, pad]).reshape(NW, N_CHUNKS, CHUNK)
    dst3 = jnp.concatenate([dst, pad]).reshape(NW, N_CHUNKS, CHUNK)
    x_pad = jnp.pad(x, ((0, N_PAD - N_NODES), (0, 0)))
    zeros1 = jnp.zeros((N_PAD,), jnp.float32)
    zeros2 = jnp.zeros((N_PAD, D), jnp.float32)

    deg_parts = _sc_hist(dst3, zeros1)
    degT = deg_parts.T
    hp = _tc_scale_matmul(x_pad, W, degT)
    agg_parts = _sc_agg(hp, src3, dst3, zeros2)
    out = _tc_finish(agg_parts, hp, degT, b.reshape(1, D))
    return out[:N_NODES]


# speedup vs baseline: 16.0410x; 1.0015x over previous
import functools

import jax
import jax.numpy as jnp
from jax import lax
from jax.experimental import pallas as pl
from jax.experimental.pallas import tpu as pltpu
from jax.experimental.pallas import tpu_sc as plsc

N_NODES = 10000
N_EDGES = 320000
D = 128

NC = 2
NS = 16
NW = NC * NS
N_PAD = 10240
E_PAD = 327680
E_PER_W = E_PAD // NW
CHUNK = 128
N_CHUNKS = E_PER_W // CHUNK
HALF = N_CHUNKS // 2
ROWS_PER_TILE = N_PAD // NS


def _sc_mesh():
    return plsc.VectorSubcoreMesh(core_axis_name="c", subcore_axis_name="s")


def _sc_hist(dst3, zeros1):
    @functools.partial(
        pl.kernel,
        out_type=jax.ShapeDtypeStruct((NC, N_PAD), jnp.float32),
        mesh=_sc_mesh(),
        scratch_types=[
            pltpu.VMEM((N_CHUNKS, CHUNK), jnp.int32),
            pltpu.VMEM((CHUNK,), jnp.float32),
            pltpu.VMEM_SHARED((N_PAD,), jnp.float32),
        ],
    )
    def hist_kernel(dst_hbm, zeros_hbm, out_hbm, idx_v, ones_v, deg_sh):
        c = lax.axis_index("c")
        s = lax.axis_index("s")
        wid = c * NS + s
        pltpu.sync_copy(
            zeros_hbm.at[pl.ds(s * ROWS_PER_TILE, ROWS_PER_TILE)],
            deg_sh.at[pl.ds(s * ROWS_PER_TILE, ROWS_PER_TILE)],
        )
        pltpu.sync_copy(dst_hbm.at[wid], idx_v)
        for i in range(CHUNK // 16):
            ones_v[pl.ds(i * 16, 16)] = jnp.ones((16,), jnp.float32)
        plsc.subcore_barrier()

        def body(j, carry):
            pltpu.sync_copy(ones_v, deg_sh.at[idx_v.at[j]], add=True)
            return carry

        lax.fori_loop(0, N_CHUNKS, body, 0)
        plsc.subcore_barrier()
        pltpu.sync_copy(
            deg_sh.at[pl.ds(s * ROWS_PER_TILE, ROWS_PER_TILE)],
            out_hbm.at[c, pl.ds(s * ROWS_PER_TILE, ROWS_PER_TILE)],
        )

    return hist_kernel(dst3, zeros1)


def _tc_scale_matmul(x_pad, W, degT):
    blk = 1024
    grid = N_PAD // blk

    def body(x_ref, w_ref, dp_ref, o_ref):
        deg = jnp.sum(dp_ref[...], axis=1, keepdims=True) + 1.0
        dis = lax.rsqrt(deg)
        h = jnp.dot(x_ref[...], w_ref[...], preferred_element_type=jnp.float32)
        o_ref[...] = dis * h

    return pl.pallas_call(
        body,
        grid=(grid,),
        in_specs=[
            pl.BlockSpec((blk, D), lambda i: (i, 0)),
            pl.BlockSpec((D, D), lambda i: (0, 0)),
            pl.BlockSpec((blk, NC), lambda i: (i, 0)),
        ],
        out_specs=pl.BlockSpec((blk, D), lambda i: (i, 0)),
        out_shape=jax.ShapeDtypeStruct((N_PAD, D), jnp.float32),
    )(x_pad, W, degT)


def _sc_agg(hp, src3, dst3, zeros2):
    @functools.partial(
        pl.kernel,
        out_type=jax.ShapeDtypeStruct((NC, N_PAD, D), jnp.float32),
        mesh=_sc_mesh(),
        scratch_types=[
            pltpu.VMEM((HALF, CHUNK), jnp.int32),
            pltpu.VMEM((HALF, CHUNK), jnp.int32),
            pltpu.VMEM((2, CHUNK, D), jnp.float32),
            pltpu.VMEM_SHARED((N_PAD, D), jnp.float32),
            pltpu.SemaphoreType.DMA,
            pltpu.SemaphoreType.DMA,
        ],
    )
    def agg_kernel(hp_hbm, src_hbm, dst_hbm, zeros_hbm, out_hbm,
                   src_v, dst_v, rows_v, agg_sh, sem0, sem1):
        c = lax.axis_index("c")
        s = lax.axis_index("s")
        wid = c * NS + s
        pltpu.sync_copy(
            zeros_hbm.at[pl.ds(s * ROWS_PER_TILE, ROWS_PER_TILE)],
            agg_sh.at[pl.ds(s * ROWS_PER_TILE, ROWS_PER_TILE)],
        )
        plsc.subcore_barrier()

        sems = (sem0, sem1)
        for half in range(2):
            pltpu.sync_copy(src_hbm.at[wid, pl.ds(half * HALF, HALF)], src_v)
            pltpu.sync_copy(dst_hbm.at[wid, pl.ds(half * HALF, HALF)], dst_v)
            pltpu.async_copy(hp_hbm.at[src_v.at[0]], rows_v.at[0], sem0)
            pltpu.async_copy(hp_hbm.at[src_v.at[1]], rows_v.at[1], sem1)

            def pair(jb, carry):
                for buf in range(2):
                    j = 2 * jb + buf
                    pltpu.make_async_copy(hp_hbm.at[src_v.at[j]],
                                          rows_v.at[buf], sems[buf]).wait()
                    pltpu.sync_copy(rows_v.at[buf], agg_sh.at[dst_v.at[j]],
                                    add=True)
                    pltpu.async_copy(hp_hbm.at[src_v.at[j + 2]],
                                     rows_v.at[buf], sems[buf])
                return carry

            lax.fori_loop(0, HALF // 2 - 1, pair, 0)
            for buf in range(2):
                j = HALF - 2 + buf
                pltpu.make_async_copy(hp_hbm.at[src_v.at[j]],
                                      rows_v.at[buf], sems[buf]).wait()
                pltpu.sync_copy(rows_v.at[buf], agg_sh.at[dst_v.at[j]],
                                add=True)

        plsc.subcore_barrier()
        pltpu.sync_copy(
            agg_sh.at[pl.ds(s * ROWS_PER_TILE, ROWS_PER_TILE)],
            out_hbm.at[c, pl.ds(s * ROWS_PER_TILE, ROWS_PER_TILE)],
        )

    return agg_kernel(hp, src3, dst3, zeros2)


def _tc_finish(agg_parts, hp, degT, b2):
    blk = 1024
    grid = N_PAD // blk

    def body(a_ref, h_ref, dp_ref, b_ref, o_ref):
        deg = jnp.sum(dp_ref[...], axis=1, keepdims=True) + 1.0
        dis = lax.rsqrt(deg)
        total = a_ref[0] + a_ref[1] + h_ref[...]
        o_ref[...] = jnp.maximum(dis * total + b_ref[...], 0.0)

    return pl.pallas_call(
        body,
        grid=(grid,),
        in_specs=[
            pl.BlockSpec((NC, blk, D), lambda i: (0, i, 0)),
            pl.BlockSpec((blk, D), lambda i: (i, 0)),
            pl.BlockSpec((blk, NC), lambda i: (i, 0)),
            pl.BlockSpec((1, D), lambda i: (0, 0)),
        ],
        out_specs=pl.BlockSpec((blk, D), lambda i: (i, 0)),
        out_shape=jax.ShapeDtypeStruct((N_PAD, D), jnp.float32),
    )(agg_parts, hp, degT, b2)


def kernel(x, edge_index, W, b):
    src = edge_index[0].astype(jnp.int32)
    dst = edge_index[1].astype(jnp.int32)
    n_dummy = E_PAD - N_EDGES
    pad_src = jnp.full((n_dummy,), N_NODES, jnp.int32)
    pad_dst = N_NODES + (jnp.arange(n_dummy, dtype=jnp.int32)
                         % (N_PAD - N_NODES))
    src3 = jnp.concatenate([src, pad_src]).reshape(NW, N_CHUNKS, CHUNK)
    dst3 = jnp.concatenate([dst, pad_dst]).reshape(NW, N_CHUNKS, CHUNK)
    x_pad = jnp.pad(x, ((0, N_PAD - N_NODES), (0, 0)))
    zeros1 = jnp.zeros((N_PAD,), jnp.float32)
    zeros2 = jnp.zeros((N_PAD, D), jnp.float32)

    deg_parts = _sc_hist(dst3, zeros1)
    degT = deg_parts.T
    hp = _tc_scale_matmul(x_pad, W, degT)
    agg_parts = _sc_agg(hp, src3, dst3, zeros2)
    out = _tc_finish(agg_parts, hp, degT, b.reshape(1, D))
    return out[:N_NODES]


# speedup vs baseline: 16.6013x; 1.0349x over previous
import functools

import jax
import jax.numpy as jnp
from jax import lax
from jax.experimental import pallas as pl
from jax.experimental.pallas import tpu as pltpu
from jax.experimental.pallas import tpu_sc as plsc

N_NODES = 10000
N_EDGES = 320000
D = 128

NC = 2
NS = 16
NW = NC * NS
N_PAD = 10240
E_PAD = 327680
E_PER_W = E_PAD // NW
CHUNK = 128
N_CHUNKS = E_PER_W // CHUNK
TOT_CHUNKS = E_PAD // CHUNK
C0_CHUNKS = 128
C1_CHUNKS = 32
C0_STAGE = 32
IDX_BUF = 32
ROWS_PER_TILE = N_PAD // NS


def _sc_mesh():
    return plsc.VectorSubcoreMesh(core_axis_name="c", subcore_axis_name="s")


def _sc_hist(dst2, zeros1):
    @functools.partial(
        pl.kernel,
        out_type=jax.ShapeDtypeStruct((NC, N_PAD), jnp.float32),
        mesh=_sc_mesh(),
        scratch_types=[
            pltpu.VMEM((N_CHUNKS, CHUNK), jnp.int32),
            pltpu.VMEM((CHUNK,), jnp.float32),
            pltpu.VMEM_SHARED((N_PAD,), jnp.float32),
        ],
    )
    def hist_kernel(dst_hbm, zeros_hbm, out_hbm, idx_v, ones_v, deg_sh):
        c = lax.axis_index("c")
        s = lax.axis_index("s")
        wid = c * NS + s
        pltpu.sync_copy(
            zeros_hbm.at[pl.ds(s * ROWS_PER_TILE, ROWS_PER_TILE)],
            deg_sh.at[pl.ds(s * ROWS_PER_TILE, ROWS_PER_TILE)],
        )
        pltpu.sync_copy(dst_hbm.at[pl.ds(wid * N_CHUNKS, N_CHUNKS)], idx_v)
        for i in range(CHUNK // 16):
            ones_v[pl.ds(i * 16, 16)] = jnp.ones((16,), jnp.float32)
        plsc.subcore_barrier()

        def body(j, carry):
            pltpu.sync_copy(ones_v, deg_sh.at[idx_v.at[j]], add=True)
            return carry

        lax.fori_loop(0, N_CHUNKS, body, 0)
        plsc.subcore_barrier()
        pltpu.sync_copy(
            deg_sh.at[pl.ds(s * ROWS_PER_TILE, ROWS_PER_TILE)],
            out_hbm.at[c, pl.ds(s * ROWS_PER_TILE, ROWS_PER_TILE)],
        )

    return hist_kernel(dst2, zeros1)


def _tc_scale_matmul(x_pad, W, degT):
    blk = 1024
    grid = N_PAD // blk

    def body(x_ref, w_ref, dp_ref, o_ref):
        deg = jnp.sum(dp_ref[...], axis=1, keepdims=True) + 1.0
        dis = lax.rsqrt(deg)
        h = jnp.dot(x_ref[...], w_ref[...], preferred_element_type=jnp.float32)
        o_ref[...] = dis * h

    return pl.pallas_call(
        body,
        grid=(grid,),
        in_specs=[
            pl.BlockSpec((blk, D), lambda i: (i, 0)),
            pl.BlockSpec((D, D), lambda i: (0, 0)),
            pl.BlockSpec((blk, NC), lambda i: (i, 0)),
        ],
        out_specs=pl.BlockSpec((blk, D), lambda i: (i, 0)),
        out_shape=jax.ShapeDtypeStruct((N_PAD, D), jnp.float32),
    )(x_pad, W, degT)


def _sc_agg(hp, src2, dst2, zeros2):
    @functools.partial(
        pl.kernel,
        out_type=jax.ShapeDtypeStruct((NC, N_PAD, D), jnp.float32),
        mesh=_sc_mesh(),
        scratch_types=[
            pltpu.VMEM((IDX_BUF, CHUNK), jnp.int32),
            pltpu.VMEM((IDX_BUF, CHUNK), jnp.int32),
            pltpu.VMEM((2, CHUNK, D), jnp.float32),
            pltpu.VMEM_SHARED((N_PAD, D), jnp.float32),
            pltpu.SemaphoreType.DMA,
            pltpu.SemaphoreType.DMA,
        ],
    )
    def agg_kernel(hp_hbm, src_hbm, dst_hbm, zeros_hbm, out_hbm,
                   src_v, dst_v, rows_v, agg_sh, sem0, sem1):
        c = lax.axis_index("c")
        s = lax.axis_index("s")
        pltpu.sync_copy(
            zeros_hbm.at[pl.ds(s * ROWS_PER_TILE, ROWS_PER_TILE)],
            agg_sh.at[pl.ds(s * ROWS_PER_TILE, ROWS_PER_TILE)],
        )
        plsc.subcore_barrier()

        sems = (sem0, sem1)

        def stage(base, n):
            pltpu.sync_copy(src_hbm.at[pl.ds(base, n)], src_v.at[pl.ds(0, n)])
            pltpu.sync_copy(dst_hbm.at[pl.ds(base, n)], dst_v.at[pl.ds(0, n)])
            pltpu.async_copy(hp_hbm.at[src_v.at[0]], rows_v.at[0], sem0)
            pltpu.async_copy(hp_hbm.at[src_v.at[1]], rows_v.at[1], sem1)

            def pair(jb, carry):
                for buf in range(2):
                    j = 2 * jb + buf
                    pltpu.make_async_copy(hp_hbm.at[src_v.at[j]],
                                          rows_v.at[buf], sems[buf]).wait()
                    pltpu.sync_copy(rows_v.at[buf], agg_sh.at[dst_v.at[j]],
                                    add=True)
                    pltpu.async_copy(hp_hbm.at[src_v.at[j + 2]],
                                     rows_v.at[buf], sems[buf])
                return carry

            lax.fori_loop(0, n // 2 - 1, pair, 0)
            for buf in range(2):
                j = n - 2 + buf
                pltpu.make_async_copy(hp_hbm.at[src_v.at[j]],
                                      rows_v.at[buf], sems[buf]).wait()
                pltpu.sync_copy(rows_v.at[buf], agg_sh.at[dst_v.at[j]],
                                add=True)

        @pl.when(c == 0)
        def _():
            for st in range(C0_CHUNKS // C0_STAGE):
                stage(s * C0_CHUNKS + st * C0_STAGE, C0_STAGE)

        @pl.when(c == 1)
        def _():
            stage(NS * C0_CHUNKS + s * C1_CHUNKS, C1_CHUNKS)

        plsc.subcore_barrier()
        pltpu.sync_copy(
            agg_sh.at[pl.ds(s * ROWS_PER_TILE, ROWS_PER_TILE)],
            out_hbm.at[c, pl.ds(s * ROWS_PER_TILE, ROWS_PER_TILE)],
        )

    return agg_kernel(hp, src2, dst2, zeros2)


def _tc_finish(agg_parts, hp, degT, b2):
    blk = 1024
    grid = N_PAD // blk

    def body(a_ref, h_ref, dp_ref, b_ref, o_ref):
        deg = jnp.sum(dp_ref[...], axis=1, keepdims=True) + 1.0
        dis = lax.rsqrt(deg)
        total = a_ref[0] + a_ref[1] + h_ref[...]
        o_ref[...] = jnp.maximum(dis * total + b_ref[...], 0.0)

    return pl.pallas_call(
        body,
        grid=(grid,),
        in_specs=[
            pl.BlockSpec((NC, blk, D), lambda i: (0, i, 0)),
            pl.BlockSpec((blk, D), lambda i: (i, 0)),
            pl.BlockSpec((blk, NC), lambda i: (i, 0)),
            pl.BlockSpec((1, D), lambda i: (0, 0)),
        ],
        out_specs=pl.BlockSpec((blk, D), lambda i: (i, 0)),
        out_shape=jax.ShapeDtypeStruct((N_PAD, D), jnp.float32),
    )(agg_parts, hp, degT, b2)


def kernel(x, edge_index, W, b):
    src = edge_index[0].astype(jnp.int32)
    dst = edge_index[1].astype(jnp.int32)
    n_dummy = E_PAD - N_EDGES
    pad_src = jnp.full((n_dummy,), N_NODES, jnp.int32)
    pad_dst = N_NODES + (jnp.arange(n_dummy, dtype=jnp.int32)
                         % (N_PAD - N_NODES))
    src2 = jnp.concatenate([src, pad_src]).reshape(TOT_CHUNKS, CHUNK)
    dst2 = jnp.concatenate([dst, pad_dst]).reshape(TOT_CHUNKS, CHUNK)
    x_pad = jnp.pad(x, ((0, N_PAD - N_NODES), (0, 0)))
    zeros1 = jnp.zeros((N_PAD,), jnp.float32)
    zeros2 = jnp.zeros((N_PAD, D), jnp.float32)

    deg_parts = _sc_hist(dst2, zeros1)
    degT = deg_parts.T
    hp = _tc_scale_matmul(x_pad, W, degT)
    agg_parts = _sc_agg(hp, src2, dst2, zeros2)
    out = _tc_finish(agg_parts, hp, degT, b.reshape(1, D))
    return out[:N_NODES]
